# baseline (speedup 1.0000x reference)
import functools

import jax
import jax.numpy as jnp
from jax import lax
from jax.experimental import pallas as pl


def _fps_body(xyz_ref, out_ref, *, n, npoint, ncols):
    x = xyz_ref[0, 0]
    y = xyz_ref[0, 1]
    z = xyz_ref[0, 2]
    rows = lax.broadcasted_iota(jnp.int32, (8, ncols), 0)
    cols = lax.broadcasted_iota(jnp.int32, (8, ncols), 1)
    org = rows * ncols + cols
    valid = org < n
    iota_np = lax.broadcasted_iota(jnp.int32, (1, npoint), 1)
    dists0 = jnp.where(valid, jnp.float32(1e10), jnp.float32(-1.0))

    def body(i, carry):
        dists, far = carry
        row = out_ref[0]
        out_ref[0] = jnp.where(iota_np == i, far, row)
        m = org == far
        cx = jnp.sum(jnp.where(m, x, 0.0))
        cy = jnp.sum(jnp.where(m, y, 0.0))
        cz = jnp.sum(jnp.where(m, z, 0.0))
        dx = x - cx
        dy = y - cy
        dz = z - cz
        d = dx * dx + dy * dy + dz * dz
        dists = jnp.where(valid, jnp.minimum(dists, d), jnp.float32(-1.0))
        mx = jnp.max(dists)
        far2 = jnp.min(jnp.where(dists == mx, org, jnp.int32(2 ** 30)))
        return dists, far2

    lax.fori_loop(0, npoint, body, (dists0, jnp.int32(0)))


def _fps(xyz, npoint):
    b, n, _ = xyz.shape
    npad = ((n + 1023) // 1024) * 1024
    ncols = npad // 8
    xt = jnp.transpose(xyz, (0, 2, 1))
    xt = jnp.pad(xt, ((0, 0), (0, 0), (0, npad - n)))
    xt = xt.reshape(b, 3, 8, ncols)
    out = pl.pallas_call(
        functools.partial(_fps_body, n=n, npoint=npoint, ncols=ncols),
        grid=(b,),
        in_specs=[pl.BlockSpec((1, 3, 8, ncols), lambda i: (i, 0, 0, 0))],
        out_specs=pl.BlockSpec((1, 1, npoint), lambda i: (i, 0, 0)),
        out_shape=jax.ShapeDtypeStruct((b, 1, npoint), jnp.int32),
    )(xt)
    return out.reshape(b, npoint)


def _mlp_max_body(x_ref, w0, b0, w1, b1, w2, b2, out_ref, *, nsample, tg):
    h = x_ref[...]
    h = jnp.maximum(jnp.dot(h, w0[...], preferred_element_type=jnp.float32)
                    + b0[...], 0.0)
    h = jnp.maximum(jnp.dot(h, w1[...], preferred_element_type=jnp.float32)
                    + b1[...], 0.0)
    h = jnp.maximum(jnp.dot(h, w2[...], preferred_element_type=jnp.float32)
                    + b2[...], 0.0)
    cout = h.shape[-1]
    out_ref[...] = jnp.max(h.reshape(tg, nsample, cout), axis=1)


def _sa_mlp_max(g, params):
    b, npnt, ns, cin = g.shape
    (w0, b0), (w1, b1), (w2, b2) = params
    cout = w2.shape[1]
    groups = b * npnt
    tg = 64
    while groups % tg:
        tg //= 2
    rows = tg * ns
    x = g.reshape(groups * ns, cin)
    wspec = lambda a: pl.BlockSpec(a.shape, lambda i: (0,) * a.ndim)
    out = pl.pallas_call(
        functools.partial(_mlp_max_body, nsample=ns, tg=tg),
        grid=(groups // tg,),
        in_specs=[pl.BlockSpec((rows, cin), lambda i: (i, 0)),
                  wspec(w0), wspec(b0.reshape(1, -1)),
                  wspec(w1), wspec(b1.reshape(1, -1)),
                  wspec(w2), wspec(b2.reshape(1, -1))],
        out_specs=pl.BlockSpec((tg, cout), lambda i: (i, 0)),
        out_shape=jax.ShapeDtypeStruct((groups, cout), jnp.float32),
    )(x, w0, b0.reshape(1, -1), w1, b1.reshape(1, -1), w2, b2.reshape(1, -1))
    return out.reshape(b, npnt, cout)


def _mlp2_body(x_ref, w0, b0, w1, b1, out_ref):
    h = x_ref[...]
    h = jnp.maximum(jnp.dot(h, w0[...], preferred_element_type=jnp.float32)
                    + b0[...], 0.0)
    h = jnp.maximum(jnp.dot(h, w1[...], preferred_element_type=jnp.float32)
                    + b1[...], 0.0)
    out_ref[...] = h


def _fp_mlp(x, params):
    b, m, cin = x.shape
    (w0, b0), (w1, b1) = params
    cout = w1.shape[1]
    rows = b * m
    tr = 512
    while rows % tr:
        tr //= 2
    wspec = lambda a: pl.BlockSpec(a.shape, lambda i: (0,) * a.ndim)
    out = pl.pallas_call(
        _mlp2_body,
        grid=(rows // tr,),
        in_specs=[pl.BlockSpec((tr, cin), lambda i: (i, 0)),
                  wspec(w0), wspec(b0.reshape(1, -1)),
                  wspec(w1), wspec(b1.reshape(1, -1))],
        out_specs=pl.BlockSpec((tr, cout), lambda i: (i, 0)),
        out_shape=jax.ShapeDtypeStruct((rows, cout), jnp.float32),
    )(x.reshape(rows, cin), w0, b0.reshape(1, -1), w1, b1.reshape(1, -1))
    return out.reshape(b, m, cout)


def _sqdist(a, b):
    a2 = jnp.sum(a * a, axis=-1)[:, :, None]
    b2 = jnp.sum(b * b, axis=-1)[:, None, :]
    ab = jnp.einsum('bnd,bmd->bnm', a, b)
    return jnp.maximum(a2 + b2 - 2.0 * ab, 0.0)


def _gather(x, idx):
    return jax.vmap(lambda a, i: a[i])(x, idx)


def _seg_select(r, k, kseg):
    *lead, nseg, segsz = r.shape
    v1, i1 = lax.top_k(r, kseg)
    gi = i1 + (jnp.arange(nseg, dtype=i1.dtype) * segsz)[:, None]
    v2, i2 = lax.top_k(v1.reshape(*lead, nseg * kseg), k)
    idx = jnp.take_along_axis(gi.reshape(*lead, nseg * kseg), i2, axis=-1)
    return v2, idx


def _topk_masked(masked, k, segsz, kseg):
    *lead, n = masked.shape
    if n % segsz:
        pad = segsz - n % segsz
        masked = jnp.pad(masked, [(0, 0)] * len(lead) + [(0, pad)],
                         constant_values=-jnp.inf)
        n += pad
    nseg = n // segsz
    r = masked.reshape(*lead, nseg, segsz)
    if kseg >= k:
        return _seg_select(r, k, kseg)
    cnt = jnp.sum(r > -jnp.inf, axis=-1)
    ok = jnp.max(cnt) <= kseg
    return lax.cond(ok,
                    lambda a: _seg_select(a, k, kseg),
                    lambda a: _seg_select(a, k, k),
                    r)


def _ball_query(radius, nsample, xyz, new_xyz):
    d2 = _sqdist(new_xyz, xyz)
    masked = jnp.where(d2 <= radius * radius, -d2, -jnp.inf)
    n = masked.shape[-1]
    if n > 4096:
        vals, idx = _topk_masked(masked, nsample, 1024, 16)
    elif n > 1024:
        vals, idx = _topk_masked(masked, nsample, 512, min(16, nsample))
    else:
        vals, idx = lax.top_k(masked, nsample)
    first = idx[..., :1]
    return jnp.where(jnp.isinf(vals), first, idx)


def _sa(xyz, feats, npoint, radius, nsample, params):
    fps_idx = _fps(xyz, npoint)
    new_xyz = _gather(xyz, fps_idx)
    idx = _ball_query(radius, nsample, xyz, new_xyz)
    g_xyz = (_gather(xyz, idx) - new_xyz[:, :, None, :]) / radius
    g = g_xyz if feats is None else jnp.concatenate(
        [g_xyz, _gather(feats, idx)], axis=-1)
    return new_xyz, _sa_mlp_max(g, params), fps_idx


def _fp(xyz1, xyz2, feats1, feats2, params):
    d2 = _sqdist(xyz1, xyz2)
    neg, idx = lax.top_k(-d2, 3)
    w = 1.0 / (jnp.maximum(-neg, 0.0) + 1e-8)
    w = w / jnp.sum(w, axis=-1, keepdims=True)
    interp = jnp.sum(_gather(feats2, idx) * w[..., None], axis=2)
    return _fp_mlp(jnp.concatenate([interp, feats1], axis=-1), params)


def _relation(xyz, feats, k=16):
    d2 = _sqdist(xyz, xyz)
    _, knn = lax.top_k(-d2, k + 1)
    knn = knn[..., 1:]
    dxyz = _gather(xyz, knn) - xyz[:, :, None, :]
    dfeat = _gather(feats, knn) - feats[:, :, None, :]
    return jnp.mean(jnp.concatenate([dxyz, dfeat], axis=-1), axis=2)


def kernel(pointcloud, sa1_w0, sa1_b0, sa1_w1, sa1_b1, sa1_w2, sa1_b2,
           sa2_w0, sa2_b0, sa2_w1, sa2_b1, sa2_w2, sa2_b2,
           sa3_w0, sa3_b0, sa3_w1, sa3_b1, sa3_w2, sa3_b2,
           sa4_w0, sa4_b0, sa4_w1, sa4_b1, sa4_w2, sa4_b2,
           fp1_w0, fp1_b0, fp1_w1, fp1_b1, fp2_w0, fp2_b0, fp2_w1, fp2_b1):
    xyz = pointcloud[..., 0:3]
    feats = pointcloud[..., 3:] if pointcloud.shape[-1] > 3 else None
    p_sa1 = [(sa1_w0, sa1_b0), (sa1_w1, sa1_b1), (sa1_w2, sa1_b2)]
    p_sa2 = [(sa2_w0, sa2_b0), (sa2_w1, sa2_b1), (sa2_w2, sa2_b2)]
    p_sa3 = [(sa3_w0, sa3_b0), (sa3_w1, sa3_b1), (sa3_w2, sa3_b2)]
    p_sa4 = [(sa4_w0, sa4_b0), (sa4_w1, sa4_b1), (sa4_w2, sa4_b2)]
    p_fp1 = [(fp1_w0, fp1_b0), (fp1_w1, fp1_b1)]
    p_fp2 = [(fp2_w0, fp2_b0), (fp2_w1, fp2_b1)]

    sa1_xyz, sa1_f, sa1_inds = _sa(xyz, feats, 2048, 0.2, 64, p_sa1)
    sa2_xyz, sa2_f, _ = _sa(sa1_xyz, sa1_f, 1024, 0.4, 32, p_sa2)
    sa3_xyz, sa3_f, _ = _sa(sa2_xyz, sa2_f, 512, 0.8, 16, p_sa3)
    sa4_xyz, sa4_f, _ = _sa(sa3_xyz, sa3_f, 256, 1.2, 16, p_sa4)
    fp1_f = _fp(sa3_xyz, sa4_xyz, sa3_f, sa4_f, p_fp1)
    fp2_f = _fp(sa2_xyz, sa3_xyz, sa2_f, fp1_f, p_fp2)
    fp2_inds = sa1_inds[:, 0:fp2_f.shape[1]]
    fp2_rel = _relation(sa2_xyz, fp2_f)
    return fp2_f, sa2_xyz, fp2_inds, fp2_rel


# speedup vs baseline: 1.1991x; 1.1991x over previous
import functools

import jax
import jax.numpy as jnp
from jax import lax
from jax.experimental import pallas as pl


def _fps_body(xyz_ref, out_ref, *, n, npoint, ncols):
    x = xyz_ref[0, 0]
    y = xyz_ref[0, 1]
    z = xyz_ref[0, 2]
    rows = lax.broadcasted_iota(jnp.int32, (8, ncols), 0)
    cols = lax.broadcasted_iota(jnp.int32, (8, ncols), 1)
    org = rows * ncols + cols
    valid = org < n
    iota_np = lax.broadcasted_iota(jnp.int32, (1, npoint), 1)
    dists0 = jnp.where(valid, jnp.float32(1e10), jnp.float32(-1.0))

    def body(i, carry):
        dists, far = carry
        row = out_ref[0]
        out_ref[0] = jnp.where(iota_np == i, far, row)
        m = org == far
        cx = jnp.sum(jnp.where(m, x, 0.0))
        cy = jnp.sum(jnp.where(m, y, 0.0))
        cz = jnp.sum(jnp.where(m, z, 0.0))
        dx = x - cx
        dy = y - cy
        dz = z - cz
        d = dx * dx + dy * dy + dz * dz
        dists = jnp.where(valid, jnp.minimum(dists, d), jnp.float32(-1.0))
        mx = jnp.max(dists)
        far2 = jnp.min(jnp.where(dists == mx, org, jnp.int32(2 ** 30)))
        return dists, far2

    lax.fori_loop(0, npoint, body, (dists0, jnp.int32(0)))


def _fps(xyz, npoint):
    b, n, _ = xyz.shape
    npad = ((n + 1023) // 1024) * 1024
    ncols = npad // 8
    xt = jnp.transpose(xyz, (0, 2, 1))
    xt = jnp.pad(xt, ((0, 0), (0, 0), (0, npad - n)))
    xt = xt.reshape(b, 3, 8, ncols)
    out = pl.pallas_call(
        functools.partial(_fps_body, n=n, npoint=npoint, ncols=ncols),
        grid=(b,),
        in_specs=[pl.BlockSpec((1, 3, 8, ncols), lambda i: (i, 0, 0, 0))],
        out_specs=pl.BlockSpec((1, 1, npoint), lambda i: (i, 0, 0)),
        out_shape=jax.ShapeDtypeStruct((b, 1, npoint), jnp.int32),
    )(xt)
    return out.reshape(b, npoint)


def _mlp_max_body(x_ref, w0, b0, w1, b1, w2, b2, out_ref, *, nsample, tg):
    h = x_ref[...]
    h = jnp.maximum(jnp.dot(h, w0[...], preferred_element_type=jnp.float32)
                    + b0[...], 0.0)
    h = jnp.maximum(jnp.dot(h, w1[...], preferred_element_type=jnp.float32)
                    + b1[...], 0.0)
    h = jnp.maximum(jnp.dot(h, w2[...], preferred_element_type=jnp.float32)
                    + b2[...], 0.0)
    cout = h.shape[-1]
    out_ref[...] = jnp.max(h.reshape(tg, nsample, cout), axis=1)


def _sa_mlp_max(g, params):
    b, npnt, ns, cin = g.shape
    (w0, b0), (w1, b1), (w2, b2) = params
    cout = w2.shape[1]
    groups = b * npnt
    tg = 64
    while groups % tg:
        tg //= 2
    rows = tg * ns
    x = g.reshape(groups * ns, cin)
    wspec = lambda a: pl.BlockSpec(a.shape, lambda i: (0,) * a.ndim)
    out = pl.pallas_call(
        functools.partial(_mlp_max_body, nsample=ns, tg=tg),
        grid=(groups // tg,),
        in_specs=[pl.BlockSpec((rows, cin), lambda i: (i, 0)),
                  wspec(w0), wspec(b0.reshape(1, -1)),
                  wspec(w1), wspec(b1.reshape(1, -1)),
                  wspec(w2), wspec(b2.reshape(1, -1))],
        out_specs=pl.BlockSpec((tg, cout), lambda i: (i, 0)),
        out_shape=jax.ShapeDtypeStruct((groups, cout), jnp.float32),
    )(x, w0, b0.reshape(1, -1), w1, b1.reshape(1, -1), w2, b2.reshape(1, -1))
    return out.reshape(b, npnt, cout)


def _mlp2_body(x_ref, w0, b0, w1, b1, out_ref):
    h = x_ref[...]
    h = jnp.maximum(jnp.dot(h, w0[...], preferred_element_type=jnp.float32)
                    + b0[...], 0.0)
    h = jnp.maximum(jnp.dot(h, w1[...], preferred_element_type=jnp.float32)
                    + b1[...], 0.0)
    out_ref[...] = h


def _fp_mlp(x, params):
    b, m, cin = x.shape
    (w0, b0), (w1, b1) = params
    cout = w1.shape[1]
    rows = b * m
    tr = 512
    while rows % tr:
        tr //= 2
    wspec = lambda a: pl.BlockSpec(a.shape, lambda i: (0,) * a.ndim)
    out = pl.pallas_call(
        _mlp2_body,
        grid=(rows // tr,),
        in_specs=[pl.BlockSpec((tr, cin), lambda i: (i, 0)),
                  wspec(w0), wspec(b0.reshape(1, -1)),
                  wspec(w1), wspec(b1.reshape(1, -1))],
        out_specs=pl.BlockSpec((tr, cout), lambda i: (i, 0)),
        out_shape=jax.ShapeDtypeStruct((rows, cout), jnp.float32),
    )(x.reshape(rows, cin), w0, b0.reshape(1, -1), w1, b1.reshape(1, -1))
    return out.reshape(b, m, cout)


def _sqdist(a, b):
    a2 = jnp.sum(a * a, axis=-1)[:, :, None]
    b2 = jnp.sum(b * b, axis=-1)[:, None, :]
    ab = jnp.einsum('bnd,bmd->bnm', a, b)
    return jnp.maximum(a2 + b2 - 2.0 * ab, 0.0)


def _gather(x, idx):
    return jax.vmap(lambda a, i: a[i])(x, idx)


def _seg_select(r, k, kseg):
    *lead, nseg, segsz = r.shape
    v1, i1 = lax.top_k(r, kseg)
    gi = i1 + (jnp.arange(nseg, dtype=i1.dtype) * segsz)[:, None]
    v2, i2 = lax.top_k(v1.reshape(*lead, nseg * kseg), k)
    idx = jnp.take_along_axis(gi.reshape(*lead, nseg * kseg), i2, axis=-1)
    return v2, idx


def _topk_masked(masked, k, segsz, kseg):
    *lead, n = masked.shape
    if n % segsz:
        pad = segsz - n % segsz
        masked = jnp.pad(masked, [(0, 0)] * len(lead) + [(0, pad)],
                         constant_values=-jnp.inf)
        n += pad
    nseg = n // segsz
    r = masked.reshape(*lead, nseg, segsz)
    if kseg >= k:
        return _seg_select(r, k, kseg)
    cnt = jnp.sum(r > -jnp.inf, axis=-1)
    ok = jnp.max(cnt) <= kseg
    return lax.cond(ok,
                    lambda a: _seg_select(a, k, kseg),
                    lambda a: _seg_select(a, k, k),
                    r)


def _ball_query(radius, nsample, xyz, new_xyz):
    d2 = _sqdist(new_xyz, xyz)
    masked = jnp.where(d2 <= radius * radius, -d2, -jnp.inf)
    n = masked.shape[-1]
    if n > 4096:
        vals, idx = _topk_masked(masked, nsample, 256, 8)
    elif n > 1024:
        vals, idx = _topk_masked(masked, nsample, 256, 8)
    else:
        vals, idx = lax.top_k(masked, nsample)
    first = idx[..., :1]
    return jnp.where(jnp.isinf(vals), first, idx)


def _sa(xyz, feats, npoint, radius, nsample, params):
    fps_idx = _fps(xyz, npoint)
    new_xyz = _gather(xyz, fps_idx)
    idx = _ball_query(radius, nsample, xyz, new_xyz)
    g_xyz = (_gather(xyz, idx) - new_xyz[:, :, None, :]) / radius
    g = g_xyz if feats is None else jnp.concatenate(
        [g_xyz, _gather(feats, idx)], axis=-1)
    return new_xyz, _sa_mlp_max(g, params), fps_idx


def _fp(xyz1, xyz2, feats1, feats2, params):
    d2 = _sqdist(xyz1, xyz2)
    neg, idx = lax.top_k(-d2, 3)
    w = 1.0 / (jnp.maximum(-neg, 0.0) + 1e-8)
    w = w / jnp.sum(w, axis=-1, keepdims=True)
    interp = jnp.sum(_gather(feats2, idx) * w[..., None], axis=2)
    return _fp_mlp(jnp.concatenate([interp, feats1], axis=-1), params)


def _relation(xyz, feats, k=16):
    d2 = _sqdist(xyz, xyz)
    _, knn = lax.top_k(-d2, k + 1)
    knn = knn[..., 1:]
    dxyz = _gather(xyz, knn) - xyz[:, :, None, :]
    dfeat = _gather(feats, knn) - feats[:, :, None, :]
    return jnp.mean(jnp.concatenate([dxyz, dfeat], axis=-1), axis=2)


def kernel(pointcloud, sa1_w0, sa1_b0, sa1_w1, sa1_b1, sa1_w2, sa1_b2,
           sa2_w0, sa2_b0, sa2_w1, sa2_b1, sa2_w2, sa2_b2,
           sa3_w0, sa3_b0, sa3_w1, sa3_b1, sa3_w2, sa3_b2,
           sa4_w0, sa4_b0, sa4_w1, sa4_b1, sa4_w2, sa4_b2,
           fp1_w0, fp1_b0, fp1_w1, fp1_b1, fp2_w0, fp2_b0, fp2_w1, fp2_b1):
    xyz = pointcloud[..., 0:3]
    feats = pointcloud[..., 3:] if pointcloud.shape[-1] > 3 else None
    p_sa1 = [(sa1_w0, sa1_b0), (sa1_w1, sa1_b1), (sa1_w2, sa1_b2)]
    p_sa2 = [(sa2_w0, sa2_b0), (sa2_w1, sa2_b1), (sa2_w2, sa2_b2)]
    p_sa3 = [(sa3_w0, sa3_b0), (sa3_w1, sa3_b1), (sa3_w2, sa3_b2)]
    p_sa4 = [(sa4_w0, sa4_b0), (sa4_w1, sa4_b1), (sa4_w2, sa4_b2)]
    p_fp1 = [(fp1_w0, fp1_b0), (fp1_w1, fp1_b1)]
    p_fp2 = [(fp2_w0, fp2_b0), (fp2_w1, fp2_b1)]

    sa1_xyz, sa1_f, sa1_inds = _sa(xyz, feats, 2048, 0.2, 64, p_sa1)
    sa2_xyz, sa2_f, _ = _sa(sa1_xyz, sa1_f, 1024, 0.4, 32, p_sa2)
    sa3_xyz, sa3_f, _ = _sa(sa2_xyz, sa2_f, 512, 0.8, 16, p_sa3)
    sa4_xyz, sa4_f, _ = _sa(sa3_xyz, sa3_f, 256, 1.2, 16, p_sa4)
    fp1_f = _fp(sa3_xyz, sa4_xyz, sa3_f, sa4_f, p_fp1)
    fp2_f = _fp(sa2_xyz, sa3_xyz, sa2_f, fp1_f, p_fp2)
    fp2_inds = sa1_inds[:, 0:fp2_f.shape[1]]
    fp2_rel = _relation(sa2_xyz, fp2_f)
    return fp2_f, sa2_xyz, fp2_inds, fp2_rel


# speedup vs baseline: 1.3467x; 1.1231x over previous
import functools

import jax
import jax.numpy as jnp
from jax import lax
from jax.experimental import pallas as pl


def _fps_body(xyz_ref, out_ref, *, n, npoint, ncols):
    x = xyz_ref[0, 0]
    y = xyz_ref[0, 1]
    z = xyz_ref[0, 2]
    rows = lax.broadcasted_iota(jnp.int32, (8, ncols), 0)
    cols = lax.broadcasted_iota(jnp.int32, (8, ncols), 1)
    org = rows * ncols + cols
    valid = org < n
    iota_np = lax.broadcasted_iota(jnp.int32, (1, npoint), 1)
    dists0 = jnp.where(valid, jnp.float32(1e10), jnp.float32(-1.0))

    def body(i, carry):
        dists, far = carry
        row = out_ref[0]
        out_ref[0] = jnp.where(iota_np == i, far, row)
        m = org == far
        cx = jnp.sum(jnp.where(m, x, 0.0))
        cy = jnp.sum(jnp.where(m, y, 0.0))
        cz = jnp.sum(jnp.where(m, z, 0.0))
        dx = x - cx
        dy = y - cy
        dz = z - cz
        d = dx * dx + dy * dy + dz * dz
        dists = jnp.where(valid, jnp.minimum(dists, d), jnp.float32(-1.0))
        mx = jnp.max(dists)
        far2 = jnp.min(jnp.where(dists == mx, org, jnp.int32(2 ** 30)))
        return dists, far2

    lax.fori_loop(0, npoint, body, (dists0, jnp.int32(0)))


def _fps(xyz, npoint):
    b, n, _ = xyz.shape
    npad = ((n + 1023) // 1024) * 1024
    ncols = npad // 8
    xt = jnp.transpose(xyz, (0, 2, 1))
    xt = jnp.pad(xt, ((0, 0), (0, 0), (0, npad - n)))
    xt = xt.reshape(b, 3, 8, ncols)
    out = pl.pallas_call(
        functools.partial(_fps_body, n=n, npoint=npoint, ncols=ncols),
        grid=(b,),
        in_specs=[pl.BlockSpec((1, 3, 8, ncols), lambda i: (i, 0, 0, 0))],
        out_specs=pl.BlockSpec((1, 1, npoint), lambda i: (i, 0, 0)),
        out_shape=jax.ShapeDtypeStruct((b, 1, npoint), jnp.int32),
    )(xt)
    return out.reshape(b, npoint)


def _mlp_max_body(x_ref, w0, b0, w1, b1, w2, b2, out_ref, *, nsample, tg):
    h = x_ref[...]
    h = jnp.maximum(jnp.dot(h, w0[...], preferred_element_type=jnp.float32)
                    + b0[...], 0.0)
    h = jnp.maximum(jnp.dot(h, w1[...], preferred_element_type=jnp.float32)
                    + b1[...], 0.0)
    h = jnp.maximum(jnp.dot(h, w2[...], preferred_element_type=jnp.float32)
                    + b2[...], 0.0)
    cout = h.shape[-1]
    out_ref[...] = jnp.max(h.reshape(tg, nsample, cout), axis=1)


def _sa_mlp_max(g, params):
    b, npnt, ns, cin = g.shape
    (w0, b0), (w1, b1), (w2, b2) = params
    cout = w2.shape[1]
    groups = b * npnt
    tg = 64
    while groups % tg:
        tg //= 2
    rows = tg * ns
    x = g.reshape(groups * ns, cin)
    wspec = lambda a: pl.BlockSpec(a.shape, lambda i: (0,) * a.ndim)
    out = pl.pallas_call(
        functools.partial(_mlp_max_body, nsample=ns, tg=tg),
        grid=(groups // tg,),
        in_specs=[pl.BlockSpec((rows, cin), lambda i: (i, 0)),
                  wspec(w0), wspec(b0.reshape(1, -1)),
                  wspec(w1), wspec(b1.reshape(1, -1)),
                  wspec(w2), wspec(b2.reshape(1, -1))],
        out_specs=pl.BlockSpec((tg, cout), lambda i: (i, 0)),
        out_shape=jax.ShapeDtypeStruct((groups, cout), jnp.float32),
    )(x, w0, b0.reshape(1, -1), w1, b1.reshape(1, -1), w2, b2.reshape(1, -1))
    return out.reshape(b, npnt, cout)


def _mlp2_body(x_ref, w0, b0, w1, b1, out_ref):
    h = x_ref[...]
    h = jnp.maximum(jnp.dot(h, w0[...], preferred_element_type=jnp.float32)
                    + b0[...], 0.0)
    h = jnp.maximum(jnp.dot(h, w1[...], preferred_element_type=jnp.float32)
                    + b1[...], 0.0)
    out_ref[...] = h


def _fp_mlp(x, params):
    b, m, cin = x.shape
    (w0, b0), (w1, b1) = params
    cout = w1.shape[1]
    rows = b * m
    tr = 512
    while rows % tr:
        tr //= 2
    wspec = lambda a: pl.BlockSpec(a.shape, lambda i: (0,) * a.ndim)
    out = pl.pallas_call(
        _mlp2_body,
        grid=(rows // tr,),
        in_specs=[pl.BlockSpec((tr, cin), lambda i: (i, 0)),
                  wspec(w0), wspec(b0.reshape(1, -1)),
                  wspec(w1), wspec(b1.reshape(1, -1))],
        out_specs=pl.BlockSpec((tr, cout), lambda i: (i, 0)),
        out_shape=jax.ShapeDtypeStruct((rows, cout), jnp.float32),
    )(x.reshape(rows, cin), w0, b0.reshape(1, -1), w1, b1.reshape(1, -1))
    return out.reshape(b, m, cout)


def _sqdist(a, b):
    a2 = jnp.sum(a * a, axis=-1)[:, :, None]
    b2 = jnp.sum(b * b, axis=-1)[:, None, :]
    ab = jnp.einsum('bnd,bmd->bnm', a, b)
    return jnp.maximum(a2 + b2 - 2.0 * ab, 0.0)


def _gather(x, idx):
    return jax.vmap(lambda a, i: a[i])(x, idx)


def _seg_select(r, k, kseg):
    *lead, nseg, segsz = r.shape
    v1, i1 = lax.top_k(r, kseg)
    gi = i1 + (jnp.arange(nseg, dtype=i1.dtype) * segsz)[:, None]
    v2, i2 = lax.top_k(v1.reshape(*lead, nseg * kseg), k)
    idx = jnp.take_along_axis(gi.reshape(*lead, nseg * kseg), i2, axis=-1)
    return v2, idx


def _topk_masked(masked, k, segsz, kseg):
    *lead, n = masked.shape
    if n % segsz:
        pad = segsz - n % segsz
        masked = jnp.pad(masked, [(0, 0)] * len(lead) + [(0, pad)],
                         constant_values=-jnp.inf)
        n += pad
    nseg = n // segsz
    r = masked.reshape(*lead, nseg, segsz)
    if kseg >= k:
        return _seg_select(r, k, kseg)
    cnt = jnp.sum(r > -jnp.inf, axis=-1)
    ok = jnp.max(cnt) <= kseg
    return lax.cond(ok,
                    lambda a: _seg_select(a, k, kseg),
                    lambda a: _seg_select(a, k, k),
                    r)


def _ball_query(radius, nsample, xyz, new_xyz):
    d2 = _sqdist(new_xyz, xyz)
    masked = jnp.where(d2 <= radius * radius, -d2, -jnp.inf)
    n = masked.shape[-1]
    if n > 4096:
        vals, idx = _topk_masked(masked, nsample, 128, 8)
    elif n > 1024:
        vals, idx = _topk_masked(masked, nsample, 128, 8)
    else:
        vals, idx = lax.top_k(masked, nsample)
    first = idx[..., :1]
    return jnp.where(jnp.isinf(vals), first, idx)


def _sa(xyz, feats, npoint, radius, nsample, params):
    fps_idx = _fps(xyz, npoint)
    new_xyz = _gather(xyz, fps_idx)
    idx = _ball_query(radius, nsample, xyz, new_xyz)
    g_xyz = (_gather(xyz, idx) - new_xyz[:, :, None, :]) / radius
    g = g_xyz if feats is None else jnp.concatenate(
        [g_xyz, _gather(feats, idx)], axis=-1)
    return new_xyz, _sa_mlp_max(g, params), fps_idx


def _fp(xyz1, xyz2, feats1, feats2, params):
    d2 = _sqdist(xyz1, xyz2)
    neg, idx = lax.top_k(-d2, 3)
    w = 1.0 / (jnp.maximum(-neg, 0.0) + 1e-8)
    w = w / jnp.sum(w, axis=-1, keepdims=True)
    interp = jnp.sum(_gather(feats2, idx) * w[..., None], axis=2)
    return _fp_mlp(jnp.concatenate([interp, feats1], axis=-1), params)


def _relation(xyz, feats, k=16):
    d2 = _sqdist(xyz, xyz)
    _, knn = lax.top_k(-d2, k + 1)
    knn = knn[..., 1:]
    dxyz = _gather(xyz, knn) - xyz[:, :, None, :]
    dfeat = _gather(feats, knn) - feats[:, :, None, :]
    return jnp.mean(jnp.concatenate([dxyz, dfeat], axis=-1), axis=2)


def kernel(pointcloud, sa1_w0, sa1_b0, sa1_w1, sa1_b1, sa1_w2, sa1_b2,
           sa2_w0, sa2_b0, sa2_w1, sa2_b1, sa2_w2, sa2_b2,
           sa3_w0, sa3_b0, sa3_w1, sa3_b1, sa3_w2, sa3_b2,
           sa4_w0, sa4_b0, sa4_w1, sa4_b1, sa4_w2, sa4_b2,
           fp1_w0, fp1_b0, fp1_w1, fp1_b1, fp2_w0, fp2_b0, fp2_w1, fp2_b1):
    xyz = pointcloud[..., 0:3]
    feats = pointcloud[..., 3:] if pointcloud.shape[-1] > 3 else None
    p_sa1 = [(sa1_w0, sa1_b0), (sa1_w1, sa1_b1), (sa1_w2, sa1_b2)]
    p_sa2 = [(sa2_w0, sa2_b0), (sa2_w1, sa2_b1), (sa2_w2, sa2_b2)]
    p_sa3 = [(sa3_w0, sa3_b0), (sa3_w1, sa3_b1), (sa3_w2, sa3_b2)]
    p_sa4 = [(sa4_w0, sa4_b0), (sa4_w1, sa4_b1), (sa4_w2, sa4_b2)]
    p_fp1 = [(fp1_w0, fp1_b0), (fp1_w1, fp1_b1)]
    p_fp2 = [(fp2_w0, fp2_b0), (fp2_w1, fp2_b1)]

    sa1_xyz, sa1_f, sa1_inds = _sa(xyz, feats, 2048, 0.2, 64, p_sa1)
    sa2_xyz, sa2_f, _ = _sa(sa1_xyz, sa1_f, 1024, 0.4, 32, p_sa2)
    sa3_xyz, sa3_f, _ = _sa(sa2_xyz, sa2_f, 512, 0.8, 16, p_sa3)
    sa4_xyz, sa4_f, _ = _sa(sa3_xyz, sa3_f, 256, 1.2, 16, p_sa4)
    fp1_f = _fp(sa3_xyz, sa4_xyz, sa3_f, sa4_f, p_fp1)
    fp2_f = _fp(sa2_xyz, sa3_xyz, sa2_f, fp1_f, p_fp2)
    fp2_inds = sa1_inds[:, 0:fp2_f.shape[1]]
    fp2_rel = _relation(sa2_xyz, fp2_f)
    return fp2_f, sa2_xyz, fp2_inds, fp2_rel


# speedup vs baseline: 1.6322x; 1.2120x over previous
import functools

import jax
import jax.numpy as jnp
from jax import lax
from jax.experimental import pallas as pl


def _fps_body(xyz_ref, out_ref, *, n, npoint, ncols):
    x = xyz_ref[0, 0]
    y = xyz_ref[0, 1]
    z = xyz_ref[0, 2]
    rows = lax.broadcasted_iota(jnp.int32, (8, ncols), 0)
    cols = lax.broadcasted_iota(jnp.int32, (8, ncols), 1)
    org = rows * ncols + cols
    valid = org < n
    iota_np = lax.broadcasted_iota(jnp.int32, (1, npoint), 1)
    dists0 = jnp.where(valid, jnp.float32(1e10), jnp.float32(-1.0))

    def body(i, carry):
        dists, far = carry
        row = out_ref[0]
        out_ref[0] = jnp.where(iota_np == i, far, row)
        m = org == far
        cx = jnp.sum(jnp.where(m, x, 0.0))
        cy = jnp.sum(jnp.where(m, y, 0.0))
        cz = jnp.sum(jnp.where(m, z, 0.0))
        dx = x - cx
        dy = y - cy
        dz = z - cz
        d = dx * dx + dy * dy + dz * dz
        dists = jnp.where(valid, jnp.minimum(dists, d), jnp.float32(-1.0))
        mx = jnp.max(dists)
        far2 = jnp.min(jnp.where(dists == mx, org, jnp.int32(2 ** 30)))
        return dists, far2

    lax.fori_loop(0, npoint, body, (dists0, jnp.int32(0)))


def _fps(xyz, npoint):
    b, n, _ = xyz.shape
    npad = ((n + 1023) // 1024) * 1024
    ncols = npad // 8
    xt = jnp.transpose(xyz, (0, 2, 1))
    xt = jnp.pad(xt, ((0, 0), (0, 0), (0, npad - n)))
    xt = xt.reshape(b, 3, 8, ncols)
    out = pl.pallas_call(
        functools.partial(_fps_body, n=n, npoint=npoint, ncols=ncols),
        grid=(b,),
        in_specs=[pl.BlockSpec((1, 3, 8, ncols), lambda i: (i, 0, 0, 0))],
        out_specs=pl.BlockSpec((1, 1, npoint), lambda i: (i, 0, 0)),
        out_shape=jax.ShapeDtypeStruct((b, 1, npoint), jnp.int32),
    )(xt)
    return out.reshape(b, npoint)


def _mlp_max_body(x_ref, w0, b0, w1, b1, w2, b2, out_ref, *, nsample, tg):
    h = x_ref[...]
    h = jnp.maximum(jnp.dot(h, w0[...], preferred_element_type=jnp.float32)
                    + b0[...], 0.0)
    h = jnp.maximum(jnp.dot(h, w1[...], preferred_element_type=jnp.float32)
                    + b1[...], 0.0)
    h = jnp.maximum(jnp.dot(h, w2[...], preferred_element_type=jnp.float32)
                    + b2[...], 0.0)
    cout = h.shape[-1]
    out_ref[...] = jnp.max(h.reshape(tg, nsample, cout), axis=1)


def _sa_mlp_max(g, params):
    b, npnt, ns, cin = g.shape
    (w0, b0), (w1, b1), (w2, b2) = params
    cout = w2.shape[1]
    groups = b * npnt
    tg = 64
    while groups % tg:
        tg //= 2
    rows = tg * ns
    x = g.reshape(groups * ns, cin)
    wspec = lambda a: pl.BlockSpec(a.shape, lambda i: (0,) * a.ndim)
    out = pl.pallas_call(
        functools.partial(_mlp_max_body, nsample=ns, tg=tg),
        grid=(groups // tg,),
        in_specs=[pl.BlockSpec((rows, cin), lambda i: (i, 0)),
                  wspec(w0), wspec(b0.reshape(1, -1)),
                  wspec(w1), wspec(b1.reshape(1, -1)),
                  wspec(w2), wspec(b2.reshape(1, -1))],
        out_specs=pl.BlockSpec((tg, cout), lambda i: (i, 0)),
        out_shape=jax.ShapeDtypeStruct((groups, cout), jnp.float32),
    )(x, w0, b0.reshape(1, -1), w1, b1.reshape(1, -1), w2, b2.reshape(1, -1))
    return out.reshape(b, npnt, cout)


def _mlp2_body(x_ref, w0, b0, w1, b1, out_ref):
    h = x_ref[...]
    h = jnp.maximum(jnp.dot(h, w0[...], preferred_element_type=jnp.float32)
                    + b0[...], 0.0)
    h = jnp.maximum(jnp.dot(h, w1[...], preferred_element_type=jnp.float32)
                    + b1[...], 0.0)
    out_ref[...] = h


def _fp_mlp(x, params):
    b, m, cin = x.shape
    (w0, b0), (w1, b1) = params
    cout = w1.shape[1]
    rows = b * m
    tr = 512
    while rows % tr:
        tr //= 2
    wspec = lambda a: pl.BlockSpec(a.shape, lambda i: (0,) * a.ndim)
    out = pl.pallas_call(
        _mlp2_body,
        grid=(rows // tr,),
        in_specs=[pl.BlockSpec((tr, cin), lambda i: (i, 0)),
                  wspec(w0), wspec(b0.reshape(1, -1)),
                  wspec(w1), wspec(b1.reshape(1, -1))],
        out_specs=pl.BlockSpec((tr, cout), lambda i: (i, 0)),
        out_shape=jax.ShapeDtypeStruct((rows, cout), jnp.float32),
    )(x.reshape(rows, cin), w0, b0.reshape(1, -1), w1, b1.reshape(1, -1))
    return out.reshape(b, m, cout)


def _sqdist(a, b):
    a2 = jnp.sum(a * a, axis=-1)[:, :, None]
    b2 = jnp.sum(b * b, axis=-1)[:, None, :]
    ab = jnp.einsum('bnd,bmd->bnm', a, b)
    return jnp.maximum(a2 + b2 - 2.0 * ab, 0.0)


def _gather(x, idx):
    return jax.vmap(lambda a, i: a[i])(x, idx)


def _seg_select(r, k, kseg):
    *lead, nseg, segsz = r.shape
    v1, i1 = lax.top_k(r, kseg)
    gi = i1 + (jnp.arange(nseg, dtype=i1.dtype) * segsz)[:, None]
    v2, i2 = lax.top_k(v1.reshape(*lead, nseg * kseg), k)
    idx = jnp.take_along_axis(gi.reshape(*lead, nseg * kseg), i2, axis=-1)
    return v2, idx


def _topk_masked(masked, k, segsz, kseg):
    *lead, n = masked.shape
    if n % segsz:
        pad = segsz - n % segsz
        masked = jnp.pad(masked, [(0, 0)] * len(lead) + [(0, pad)],
                         constant_values=-jnp.inf)
        n += pad
    nseg = n // segsz
    r = masked.reshape(*lead, nseg, segsz)
    if kseg >= k:
        return _seg_select(r, k, kseg)
    cnt = jnp.sum(r > -jnp.inf, axis=-1)
    ok = jnp.max(cnt) <= kseg
    return lax.cond(ok,
                    lambda a: _seg_select(a, k, kseg),
                    lambda a: _seg_select(a, k, k),
                    r)


def _ball_query(radius, nsample, xyz, new_xyz):
    d2 = _sqdist(new_xyz, xyz)
    masked = jnp.where(d2 <= radius * radius, -d2, -jnp.inf)
    n = masked.shape[-1]
    if n > 4096:
        vals, idx = _topk_masked(masked, nsample, 64, 8)
    elif n > 1024:
        vals, idx = _topk_masked(masked, nsample, 64, 8)
    else:
        vals, idx = lax.top_k(masked, nsample)
    first = idx[..., :1]
    return jnp.where(jnp.isinf(vals), first, idx)


def _sa(xyz, feats, npoint, radius, nsample, params):
    fps_idx = _fps(xyz, npoint)
    new_xyz = _gather(xyz, fps_idx)
    idx = _ball_query(radius, nsample, xyz, new_xyz)
    g_xyz = (_gather(xyz, idx) - new_xyz[:, :, None, :]) / radius
    g = g_xyz if feats is None else jnp.concatenate(
        [g_xyz, _gather(feats, idx)], axis=-1)
    return new_xyz, _sa_mlp_max(g, params), fps_idx


def _fp(xyz1, xyz2, feats1, feats2, params):
    d2 = _sqdist(xyz1, xyz2)
    neg, idx = lax.top_k(-d2, 3)
    w = 1.0 / (jnp.maximum(-neg, 0.0) + 1e-8)
    w = w / jnp.sum(w, axis=-1, keepdims=True)
    interp = jnp.sum(_gather(feats2, idx) * w[..., None], axis=2)
    return _fp_mlp(jnp.concatenate([interp, feats1], axis=-1), params)


def _relation(xyz, feats, k=16):
    d2 = _sqdist(xyz, xyz)
    _, knn = lax.top_k(-d2, k + 1)
    knn = knn[..., 1:]
    dxyz = _gather(xyz, knn) - xyz[:, :, None, :]
    dfeat = _gather(feats, knn) - feats[:, :, None, :]
    return jnp.mean(jnp.concatenate([dxyz, dfeat], axis=-1), axis=2)


def kernel(pointcloud, sa1_w0, sa1_b0, sa1_w1, sa1_b1, sa1_w2, sa1_b2,
           sa2_w0, sa2_b0, sa2_w1, sa2_b1, sa2_w2, sa2_b2,
           sa3_w0, sa3_b0, sa3_w1, sa3_b1, sa3_w2, sa3_b2,
           sa4_w0, sa4_b0, sa4_w1, sa4_b1, sa4_w2, sa4_b2,
           fp1_w0, fp1_b0, fp1_w1, fp1_b1, fp2_w0, fp2_b0, fp2_w1, fp2_b1):
    xyz = pointcloud[..., 0:3]
    feats = pointcloud[..., 3:] if pointcloud.shape[-1] > 3 else None
    p_sa1 = [(sa1_w0, sa1_b0), (sa1_w1, sa1_b1), (sa1_w2, sa1_b2)]
    p_sa2 = [(sa2_w0, sa2_b0), (sa2_w1, sa2_b1), (sa2_w2, sa2_b2)]
    p_sa3 = [(sa3_w0, sa3_b0), (sa3_w1, sa3_b1), (sa3_w2, sa3_b2)]
    p_sa4 = [(sa4_w0, sa4_b0), (sa4_w1, sa4_b1), (sa4_w2, sa4_b2)]
    p_fp1 = [(fp1_w0, fp1_b0), (fp1_w1, fp1_b1)]
    p_fp2 = [(fp2_w0, fp2_b0), (fp2_w1, fp2_b1)]

    sa1_xyz, sa1_f, sa1_inds = _sa(xyz, feats, 2048, 0.2, 64, p_sa1)
    sa2_xyz, sa2_f, _ = _sa(sa1_xyz, sa1_f, 1024, 0.4, 32, p_sa2)
    sa3_xyz, sa3_f, _ = _sa(sa2_xyz, sa2_f, 512, 0.8, 16, p_sa3)
    sa4_xyz, sa4_f, _ = _sa(sa3_xyz, sa3_f, 256, 1.2, 16, p_sa4)
    fp1_f = _fp(sa3_xyz, sa4_xyz, sa3_f, sa4_f, p_fp1)
    fp2_f = _fp(sa2_xyz, sa3_xyz, sa2_f, fp1_f, p_fp2)
    fp2_inds = sa1_inds[:, 0:fp2_f.shape[1]]
    fp2_rel = _relation(sa2_xyz, fp2_f)
    return fp2_f, sa2_xyz, fp2_inds, fp2_rel
